# baseline (speedup 1.0000x reference)
import functools

import jax
import jax.numpy as jnp
from jax import lax
from jax.experimental import pallas as pl
from jax.experimental.pallas import tpu as pltpu
from jax.experimental.pallas import tpu_sc as plsc

B, L, D = 1024, 50, 512
_info = plsc.get_sparse_core_info()
_NC, _NS = _info.num_cores, _info.num_subcores
_NW = _NC * _NS
_BPW = B // _NW


@functools.partial(
    pl.kernel,
    out_type=(
        jax.ShapeDtypeStruct((B, D), jnp.float32),
        jax.ShapeDtypeStruct((B, D), jnp.float32),
    ),
    mesh=plsc.VectorSubcoreMesh(core_axis_name="c", subcore_axis_name="s"),
    scratch_types=[
        pltpu.VMEM((_BPW,), jnp.int32),
        pltpu.VMEM((_BPW,), jnp.int32),
        pltpu.VMEM((_BPW, D), jnp.float32),
        pltpu.VMEM((_BPW, D), jnp.float32),
        pltpu.SemaphoreType.DMA,
        pltpu.SemaphoreType.DMA,
    ],
)
def _sc_gather(gidx_hbm, didx_hbm, wg_hbm, wd_hbm, outg_hbm, outd_hbm,
               gidx_v, didx_v, grows_v, drows_v, sem_g, sem_d):
    wid = lax.axis_index("s") * _NC + lax.axis_index("c")
    base = wid * _BPW
    pltpu.sync_copy(gidx_hbm.at[pl.ds(base, _BPW)], gidx_v)
    pltpu.sync_copy(didx_hbm.at[pl.ds(base, _BPW)], didx_v)
    cg = pltpu.async_copy(wg_hbm.at[gidx_v], grows_v, sem_g)
    cd = pltpu.async_copy(wd_hbm.at[didx_v], drows_v, sem_d)
    cg.wait()
    cd.wait()
    pltpu.sync_copy(grows_v, outg_hbm.at[pl.ds(base, _BPW)])
    pltpu.sync_copy(drows_v, outd_hbm.at[pl.ds(base, _BPW)])


_BB = 16
_NB = 8
_C = B // _BB


def _tc_body(f_hbm, g_ref, d_ref, o_hbm, fbuf, obuf, insem, outsem):
    def start_in(i):
        pltpu.make_async_copy(f_hbm.at[pl.ds(i * _BB, _BB)],
                              fbuf.at[i % _NB], insem.at[i % _NB]).start()

    def start_out(i):
        pltpu.make_async_copy(obuf.at[i % _NB],
                              o_hbm.at[pl.ds(i * _BB, _BB)],
                              outsem.at[i % _NB]).start()

    def wait_in(i):
        pltpu.make_async_copy(f_hbm.at[pl.ds(i * _BB, _BB)],
                              fbuf.at[i % _NB], insem.at[i % _NB]).wait()

    def wait_out(i):
        pltpu.make_async_copy(obuf.at[i % _NB],
                              o_hbm.at[pl.ds(i * _BB, _BB)],
                              outsem.at[i % _NB]).wait()

    for i in range(_NB):
        start_in(i)
    for i in range(_C):
        wait_in(i)
        if i >= _NB:
            wait_out(i - _NB)
        s = i % _NB
        obuf[s, :, 2:, :] = fbuf[s]
        obuf[s, :, 0, :] = g_ref[pl.ds(i * _BB, _BB), :]
        obuf[s, :, 1, :] = d_ref[pl.ds(i * _BB, _BB), :]
        start_out(i)
        if i + _NB < _C:
            start_in(i + _NB)
    for i in range(max(_C - _NB, 0), _C):
        wait_out(i)


def _tc_assemble(feature, embg, embd):
    return pl.pallas_call(
        _tc_body,
        in_specs=[
            pl.BlockSpec(memory_space=pl.ANY),
            pl.BlockSpec((B, D), lambda: (0, 0)),
            pl.BlockSpec((B, D), lambda: (0, 0)),
        ],
        out_specs=pl.BlockSpec(memory_space=pl.ANY),
        out_shape=jax.ShapeDtypeStruct((B, L + 2, D), jnp.float32),
        scratch_shapes=[
            pltpu.VMEM((_NB, _BB, L, D), jnp.float32),
            pltpu.VMEM((_NB, _BB, L + 2, D), jnp.float32),
            pltpu.SemaphoreType.DMA((_NB,)),
            pltpu.SemaphoreType.DMA((_NB,)),
        ],
    )(feature, embg, embd)


def kernel(feature, genre_index, difficulty_index, W_genre, W_difficulty):
    gidx = genre_index.reshape(B).astype(jnp.int32)
    didx = difficulty_index.reshape(B).astype(jnp.int32)
    embg, embd = _sc_gather(gidx, didx, W_genre, W_difficulty)
    return _tc_assemble(feature, embg, embd)
